# baseline (speedup 1.0000x reference)
import functools

import jax
import jax.numpy as jnp
from jax import lax
from jax.experimental import pallas as pl
from jax.experimental.pallas import tpu as pltpu
from jax.experimental.pallas import tpu_sc as plsc

_NUM_CORES = 2
_NUM_SUBCORES = 16
_NUM_WORKERS = _NUM_CORES * _NUM_SUBCORES
_LANES = 16


def _sc_gather_t(tok3, weight):
    nw, bw, hist = tok3.shape
    batch = nw * bw
    d = weight.shape[1]
    tstride = bw + 1
    mesh = plsc.VectorSubcoreMesh(core_axis_name="c", subcore_axis_name="s")

    @functools.partial(
        pl.kernel,
        mesh=mesh,
        out_type=jax.ShapeDtypeStruct((hist, d, batch), jnp.float32),
        scratch_types=[
            pltpu.VMEM((bw, hist), jnp.int32),
            pltpu.VMEM((hist, bw), jnp.int32),
            pltpu.VMEM((2, bw, d), jnp.float32),
            pltpu.VMEM((2, d, tstride), jnp.float32),
            pltpu.SemaphoreType.DMA((2,)),
            pltpu.SemaphoreType.DMA((2,)),
        ],
        compiler_params=pltpu.CompilerParams(
            use_tc_tiling_on_sc=False, needs_layout_passes=False
        ),
    )
    def k(tok_hbm, table_hbm, out_hbm, idxb_v, idx_v, rows_v, trans_v,
          sem_g, sem_s):
        wid = lax.axis_index("s") * _NUM_CORES + lax.axis_index("c")
        b0 = wid * bw
        pltpu.sync_copy(tok_hbm.at[wid], idxb_v)
        iota0 = jnp.arange(_LANES, dtype=jnp.int32)
        for h in range(hist):
            hvec = jnp.full((_LANES,), h, dtype=jnp.int32)

            def iblk(j0, carry, hvec=hvec, h=h):
                vals = plsc.load_gather(idxb_v, [j0 + iota0, hvec])
                idx_v[h, pl.ds(pl.multiple_of(j0, _LANES), _LANES)] = vals
                return carry

            plsc.parallel_loop(0, bw, step=_LANES, unroll=2)(
                lambda j0: iblk(j0, None)
            )

        def start_gather(h, b):
            pltpu.async_copy(table_hbm.at[idx_v.at[h]], rows_v.at[b], sem_g.at[b])

        def wait_gather(h, b):
            pltpu.make_async_copy(
                table_hbm.at[idx_v.at[h]], rows_v.at[b], sem_g.at[b]
            ).wait()

        iota = jnp.arange(_LANES, dtype=jnp.int32)

        def transpose(b):
            rows = rows_v.at[b]
            trans = trans_v.at[b]

            def one_row(j, carry):
                jvec = jnp.full((_LANES,), j, dtype=jnp.int32)
                for e0 in range(d // _LANES):
                    cols = e0 * _LANES + iota
                    vals = plsc.load_gather(rows, [jvec, cols])
                    plsc.store_scatter(trans, [cols, jvec], vals)
                return carry

            plsc.parallel_loop(0, bw, unroll=8)(lambda j: one_row(j, None))

        def fire_stores(h, b):
            pltpu.async_copy(
                trans_v.at[b, :, pl.ds(0, bw)],
                out_hbm.at[h, :, pl.ds(b0, bw)],
                sem_s.at[b],
            )

        def wait_stores(h, b):
            pltpu.make_async_copy(
                trans_v.at[b, :, pl.ds(0, bw)],
                out_hbm.at[h, :, pl.ds(b0, bw)],
                sem_s.at[b],
            ).wait()

        start_gather(0, 0)
        wait_gather(0, 0)
        start_gather(1, 1)
        transpose(0)
        fire_stores(0, 0)
        wait_gather(1, 1)
        start_gather(2, 0)
        transpose(1)
        fire_stores(1, 1)

        def body(h, b):
            wait_gather(h, b)
            start_gather(h + 1, 1 - b)
            wait_stores(h - 2, b)
            transpose(b)
            fire_stores(h, b)

        def pair(g, carry):
            body(2 * g, 0)
            body(2 * g + 1, 1)
            return carry

        pl.loop(1, (hist - 2) // 2)(lambda g: pair(g, None))

        body(hist - 2, (hist - 2) % 2)

        hl = hist - 1
        bl = hl % 2
        wait_gather(hl, bl)
        wait_stores(hl - 2, bl)
        transpose(bl)
        fire_stores(hl, bl)
        wait_stores(hl - 1, 1 - bl)
        wait_stores(hl, bl)

    return k(tok3, weight)


def kernel(token_ids, weight):
    batch, hist = token_ids.shape
    tok3 = token_ids.reshape(
        _NUM_WORKERS, batch // _NUM_WORKERS, hist
    ).astype(jnp.int32)
    out_t = _sc_gather_t(tok3, weight)
    return out_t.transpose(2, 0, 1)


# speedup vs baseline: 1.0171x; 1.0171x over previous
import functools

import jax
import jax.numpy as jnp
from jax import lax
from jax.experimental import pallas as pl
from jax.experimental.pallas import tpu as pltpu
from jax.experimental.pallas import tpu_sc as plsc

_NUM_CORES = 2
_NUM_SUBCORES = 16
_NUM_WORKERS = _NUM_CORES * _NUM_SUBCORES
_LANES = 16


def _sc_gather_t(tok3, weight):
    hist, nw, bw = tok3.shape
    batch = nw * bw
    d = weight.shape[1]
    tstride = bw + 1
    mesh = plsc.VectorSubcoreMesh(core_axis_name="c", subcore_axis_name="s")

    @functools.partial(
        pl.kernel,
        mesh=mesh,
        out_type=jax.ShapeDtypeStruct((hist, d, batch), jnp.float32),
        scratch_types=[
            pltpu.VMEM((hist, bw), jnp.int32),
            pltpu.VMEM((2, bw, d), jnp.float32),
            pltpu.VMEM((2, d, tstride), jnp.float32),
            pltpu.SemaphoreType.DMA((2,)),
            pltpu.SemaphoreType.DMA((2,)),
        ],
        compiler_params=pltpu.CompilerParams(
            use_tc_tiling_on_sc=False, needs_layout_passes=False
        ),
    )
    def k(tok_hbm, table_hbm, out_hbm, idx_v, rows_v, trans_v, sem_g, sem_s):
        wid = lax.axis_index("s") * _NUM_CORES + lax.axis_index("c")
        b0 = wid * bw
        pltpu.sync_copy(tok_hbm.at[:, wid], idx_v)

        def start_gather(h, b):
            pltpu.async_copy(table_hbm.at[idx_v.at[h]], rows_v.at[b], sem_g.at[b])

        def wait_gather(h, b):
            pltpu.make_async_copy(
                table_hbm.at[idx_v.at[h]], rows_v.at[b], sem_g.at[b]
            ).wait()

        iota = jnp.arange(_LANES, dtype=jnp.int32)

        def transpose(b):
            rows = rows_v.at[b]
            trans = trans_v.at[b]

            def one_row(j, carry):
                jvec = jnp.full((_LANES,), j, dtype=jnp.int32)
                for e0 in range(d // _LANES):
                    cols = e0 * _LANES + iota
                    vals = plsc.load_gather(rows, [jvec, cols])
                    plsc.store_scatter(trans, [cols, jvec], vals)
                return carry

            plsc.parallel_loop(0, bw, unroll=8)(lambda j: one_row(j, None))

        def fire_stores(h, b):
            pltpu.async_copy(
                trans_v.at[b, :, pl.ds(0, bw)],
                out_hbm.at[h, :, pl.ds(b0, bw)],
                sem_s.at[b],
            )

        def wait_stores(h, b):
            pltpu.make_async_copy(
                trans_v.at[b, :, pl.ds(0, bw)],
                out_hbm.at[h, :, pl.ds(b0, bw)],
                sem_s.at[b],
            ).wait()

        start_gather(0, 0)
        wait_gather(0, 0)
        start_gather(1, 1)
        transpose(0)
        fire_stores(0, 0)
        wait_gather(1, 1)
        start_gather(2, 0)
        transpose(1)
        fire_stores(1, 1)

        def body(h, b):
            wait_gather(h, b)
            start_gather(h + 1, 1 - b)
            wait_stores(h - 2, b)
            transpose(b)
            fire_stores(h, b)

        def pair(g, carry):
            body(2 * g, 0)
            body(2 * g + 1, 1)
            return carry

        pl.loop(1, (hist - 2) // 2)(lambda g: pair(g, None))

        body(hist - 2, (hist - 2) % 2)

        hl = hist - 1
        bl = hl % 2
        wait_gather(hl, bl)
        wait_stores(hl - 2, bl)
        transpose(bl)
        fire_stores(hl, bl)
        wait_stores(hl - 1, 1 - bl)
        wait_stores(hl, bl)

    return k(tok3, weight)


def kernel(token_ids, weight):
    hist = token_ids.shape[1]
    tok3 = token_ids.T.reshape(hist, _NUM_WORKERS, -1).astype(jnp.int32)
    out_t = _sc_gather_t(tok3, weight)
    return out_t.transpose(2, 0, 1)
